# baseline (speedup 1.0000x reference)
import jax
import jax.numpy as jnp
from jax import lax
from jax.experimental import pallas as pl
from jax.experimental.pallas import tpu as pltpu
from jax.experimental.pallas import tpu_sc as plsc

N = 10000
NP = 10240
D = 128
E = 320000
K = 16
ALPHA = 0.05

NC = 2
NS = 16
L = 16

DH = D // NC
CH = 128
SUPC = 8
NSUP = 20
EPS = CH * SUPC * NSUP
PE = NS * EPS
RPS = NP // NS
BLK = 64
NBLK = RPS // BLK


def _broadcast_lane(wv, j):
  return lax.gather(
      wv, jnp.full((L, 1), j, jnp.int32),
      lax.GatherDimensionNumbers(
          offset_dims=(), collapsed_slice_dims=(0,), start_index_map=(0,)),
      slice_sizes=(1,),
      mode=lax.GatherScatterMode.PROMISE_IN_BOUNDS)


def _body(x2, src_e, dst_e, w_e, out2, xacc_hbm, acc, h_sp,
          sidx0, sidx1, didx0, didx1, wb0, wb1,
          rows0, rows1, rows2, rows3, ebuf, ebuf2,
          sem_g0, sem_g1, sem_g2, sem_g3,
          sem_s0, sem_s1, sem_s2, sem_s3, sem_i0, sem_i1):
  c = lax.axis_index("c")
  s = lax.axis_index("s")
  rbase = s * RPS
  zero = jnp.zeros((L,), jnp.float32)
  slots = (
      (sidx0, didx0, wb0, sem_i0),
      (sidx1, didx1, wb1, sem_i1),
  )
  rowbufs = (rows0, rows1, rows2, rows3)
  sem_g = (sem_g0, sem_g1, sem_g2, sem_g3)
  sem_s = (sem_s0, sem_s1, sem_s2, sem_s3)

  def fire_idx(u, slot):
    sidx, didx, wb, sem_i = slots[slot]
    pltpu.async_copy(src_e.at[s, u], sidx, sem_i)
    pltpu.async_copy(dst_e.at[s, u], didx, sem_i)
    pltpu.async_copy(w_e.at[s, u], wb, sem_i)

  def process(slot):
    sidx, didx, wb, sem_i = slots[slot]
    pltpu.make_async_copy(src_e.at[s, 0], sidx, sem_i).wait()
    pltpu.make_async_copy(dst_e.at[s, 0], didx, sem_i).wait()
    pltpu.make_async_copy(w_e.at[s, 0], wb, sem_i).wait()

    g = {0: pltpu.async_copy(h_sp.at[sidx.at[0]], rows0, sem_g0),
         1: pltpu.async_copy(h_sp.at[sidx.at[1]], rows1, sem_g1),
         2: pltpu.async_copy(h_sp.at[sidx.at[2]], rows2, sem_g2)}
    sc = {}
    for uu in range(SUPC):
      cur = uu % 4
      rb = rowbufs[cur]
      if uu + 3 < SUPC:
        nb = (uu + 3) % 4
        if uu >= 1:
          sc[uu - 1].wait()
        g[uu + 3] = pltpu.async_copy(
            h_sp.at[sidx.at[uu + 3]], rowbufs[nb], sem_g[nb])
      g[uu].wait()

      def scale(t, _):
        ebase = t * L
        wv = wb[uu, pl.ds(ebase, L)]
        for j in range(L):
          wbj = _broadcast_lane(wv, j)
          for q in range(DH // L):
            sl = pl.ds(q * L, L)
            rb[ebase + j, sl] = rb[ebase + j, sl] * wbj
        return 0
      lax.fori_loop(0, CH // L, scale, 0)
      sc[uu] = pltpu.async_copy(rb, acc.at[didx.at[uu]], sem_s[cur], add=True)
    sc[SUPC - 4].wait()
    sc[SUPC - 3].wait()
    sc[SUPC - 2].wait()
    sc[SUPC - 1].wait()

  def zero_ebuf(i, _):
    for j in range(DH // L):
      ebuf[i, pl.ds(j * L, L)] = zero
    return 0
  lax.fori_loop(0, BLK, zero_ebuf, 0)
  for b in range(NBLK):
    base = rbase + b * BLK
    pltpu.sync_copy(ebuf, acc.at[pl.ds(base, BLK)])
    pltpu.sync_copy(ebuf, xacc_hbm.at[c, pl.ds(base, BLK)])
    pltpu.sync_copy(x2.at[c, pl.ds(base, BLK)], ebuf2)
    pltpu.sync_copy(ebuf2, h_sp.at[pl.ds(base, BLK)])
  plsc.subcore_barrier()

  def hop(_, carry):
    fire_idx(0, 0)

    def pair(v, _):
      fire_idx(2 * v + 1, 1)
      process(0)
      fire_idx(2 * v + 2, 0)
      process(1)
      return 0
    lax.fori_loop(0, NSUP // 2 - 1, pair, 0)
    fire_idx(NSUP - 1, 1)
    process(0)
    process(1)
    plsc.subcore_barrier()

    for b in range(NBLK):
      base = rbase + b * BLK
      pltpu.sync_copy(acc.at[pl.ds(base, BLK)], ebuf)
      pltpu.sync_copy(xacc_hbm.at[c, pl.ds(base, BLK)], ebuf2)

      def scale(i, _):
        for j in range(DH // L):
          sl = pl.ds(j * L, L)
          v = ebuf[i, sl] * (1.0 - ALPHA)
          ebuf[i, sl] = v
          ebuf2[i, sl] = ebuf2[i, sl] + v
        return 0
      lax.fori_loop(0, BLK, scale, 0)
      pltpu.sync_copy(ebuf, h_sp.at[pl.ds(base, BLK)])
      pltpu.sync_copy(ebuf2, xacc_hbm.at[c, pl.ds(base, BLK)])

      def rezero(i, _):
        for j in range(DH // L):
          ebuf[i, pl.ds(j * L, L)] = zero
        return 0
      lax.fori_loop(0, BLK, rezero, 0)
      pltpu.sync_copy(ebuf, acc.at[pl.ds(base, BLK)])
    plsc.subcore_barrier()
    return carry
  lax.fori_loop(0, K, hop, 0)

  for b in range(NBLK):
    base = rbase + b * BLK
    pltpu.sync_copy(x2.at[c, pl.ds(base, BLK)], ebuf)
    pltpu.sync_copy(xacc_hbm.at[c, pl.ds(base, BLK)], ebuf2)

    def fin(i, _):
      for j in range(DH // L):
        sl = pl.ds(j * L, L)
        ebuf[i, sl] = (ebuf2[i, sl] * (1.0 / K)
                       + ebuf[i, sl] * (1.0 / K + ALPHA))
      return 0
    lax.fori_loop(0, BLK, fin, 0)
    pltpu.sync_copy(ebuf, out2.at[c, pl.ds(base, BLK)])


@jax.jit
def _run(x2, src_e, dst_e, w_e):
  mesh = plsc.VectorSubcoreMesh(core_axis_name="c", subcore_axis_name="s")
  f = pl.kernel(
      _body,
      out_type=(
          jax.ShapeDtypeStruct((NC, NP, DH), jnp.float32),
          jax.ShapeDtypeStruct((NC, NP, DH), jnp.float32),
      ),
      mesh=mesh,
      compiler_params=pltpu.CompilerParams(use_tc_tiling_on_sc=False),
      scratch_types=[
          pltpu.VMEM_SHARED((NP, DH), jnp.float32),
          pltpu.VMEM_SHARED((NP, DH), jnp.float32),
          pltpu.VMEM((SUPC, CH), jnp.int32),
          pltpu.VMEM((SUPC, CH), jnp.int32),
          pltpu.VMEM((SUPC, CH), jnp.int32),
          pltpu.VMEM((SUPC, CH), jnp.int32),
          pltpu.VMEM((SUPC, CH), jnp.float32),
          pltpu.VMEM((SUPC, CH), jnp.float32),
          pltpu.VMEM((CH, DH), jnp.float32),
          pltpu.VMEM((CH, DH), jnp.float32),
          pltpu.VMEM((CH, DH), jnp.float32),
          pltpu.VMEM((CH, DH), jnp.float32),
          pltpu.VMEM((BLK, DH), jnp.float32),
          pltpu.VMEM((BLK, DH), jnp.float32),
          pltpu.SemaphoreType.DMA,
          pltpu.SemaphoreType.DMA,
          pltpu.SemaphoreType.DMA,
          pltpu.SemaphoreType.DMA,
          pltpu.SemaphoreType.DMA,
          pltpu.SemaphoreType.DMA,
          pltpu.SemaphoreType.DMA,
          pltpu.SemaphoreType.DMA,
          pltpu.SemaphoreType.DMA,
          pltpu.SemaphoreType.DMA,
      ],
  )
  out2, _ = f(x2, src_e, dst_e, w_e)
  return out2


def kernel(x, edge_index, edge_weight):
  x = x.astype(jnp.float32)
  src = edge_index[0].astype(jnp.int32)
  dst = edge_index[1].astype(jnp.int32)
  w = edge_weight.astype(jnp.float32)
  pad = PE - E
  src_e = jnp.concatenate([src, jnp.zeros((pad,), jnp.int32)]).reshape(
      NS, NSUP, SUPC, CH)
  dst_e = jnp.concatenate([dst, jnp.zeros((pad,), jnp.int32)]).reshape(
      NS, NSUP, SUPC, CH)
  w_e = jnp.concatenate([w, jnp.zeros((pad,), jnp.float32)]).reshape(
      NS, NSUP, SUPC, CH)
  x2 = x.reshape(N, NC, DH).transpose(1, 0, 2)
  x2 = jnp.concatenate(
      [x2, jnp.zeros((NC, NP - N, DH), jnp.float32)], axis=1)
  out2 = _run(x2, src_e, dst_e, w_e)
  return out2[:, :N].transpose(1, 0, 2).reshape(N, D)


# speedup vs baseline: 2.2444x; 2.2444x over previous
import jax
import jax.numpy as jnp
from jax import lax
from jax.experimental import pallas as pl
from jax.experimental.pallas import tpu as pltpu
from jax.experimental.pallas import tpu_sc as plsc

N = 10000
NP = 10240
D = 128
E = 320000
K = 16
ALPHA = 0.05

NC = 2
NS = 16
L = 16

DH = D // NC
CH = 128
SUPC = 8
NSUP = 20
EPS = CH * SUPC * NSUP
PE = NS * EPS
RPS = NP // NS
BLK = 64
NBLK = RPS // BLK


def _broadcast_lane(wv, j):
  return lax.gather(
      wv, jnp.full((L, 1), j, jnp.int32),
      lax.GatherDimensionNumbers(
          offset_dims=(), collapsed_slice_dims=(0,), start_index_map=(0,)),
      slice_sizes=(1,),
      mode=lax.GatherScatterMode.PROMISE_IN_BOUNDS)


def _body(x2, src_e, dst_e, w_e, out2, xacc_hbm, acc, h_sp,
          sidx0, sidx1, didx0, didx1, wb0, wb1,
          rows0, rows1, rows2, rows3, ebuf, ebuf2,
          sem_g0, sem_g1, sem_g2, sem_g3,
          sem_s0, sem_s1, sem_s2, sem_s3, sem_i0, sem_i1):
  c = lax.axis_index("c")
  s = lax.axis_index("s")
  rbase = s * RPS
  zero = jnp.zeros((L,), jnp.float32)
  slots = (
      (sidx0, didx0, wb0, sem_i0),
      (sidx1, didx1, wb1, sem_i1),
  )
  rowbufs = (rows0, rows1, rows2, rows3)
  sem_g = (sem_g0, sem_g1, sem_g2, sem_g3)
  sem_s = (sem_s0, sem_s1, sem_s2, sem_s3)

  def fire_idx(u, slot):
    sidx, didx, wb, sem_i = slots[slot]
    pltpu.async_copy(src_e.at[s, u], sidx, sem_i)
    pltpu.async_copy(dst_e.at[s, u], didx, sem_i)
    pltpu.async_copy(w_e.at[s, u], wb, sem_i)

  def process(slot):
    sidx, didx, wb, sem_i = slots[slot]
    pltpu.make_async_copy(src_e.at[s, 0], sidx, sem_i).wait()
    pltpu.make_async_copy(dst_e.at[s, 0], didx, sem_i).wait()
    pltpu.make_async_copy(w_e.at[s, 0], wb, sem_i).wait()

    g = {0: pltpu.async_copy(h_sp.at[sidx.at[0]], rows0, sem_g0),
         1: pltpu.async_copy(h_sp.at[sidx.at[1]], rows1, sem_g1),
         2: pltpu.async_copy(h_sp.at[sidx.at[2]], rows2, sem_g2)}
    sc = {}
    for uu in range(SUPC):
      cur = uu % 4
      rb = rowbufs[cur]
      if uu + 3 < SUPC:
        nb = (uu + 3) % 4
        if uu >= 1:
          sc[uu - 1].wait()
        g[uu + 3] = pltpu.async_copy(
            h_sp.at[sidx.at[uu + 3]], rowbufs[nb], sem_g[nb])
      g[uu].wait()

      def scale(t, _):
        for h2 in range(2):
          ebase = t * (2 * L) + h2 * L
          wv = wb[uu, pl.ds(ebase, L)]
          for j in range(L):
            wbj = _broadcast_lane(wv, j)
            for q in range(DH // L):
              sl = pl.ds(q * L, L)
              rb[ebase + j, sl] = rb[ebase + j, sl] * wbj
        return 0
      lax.fori_loop(0, CH // (2 * L), scale, 0)
      sc[uu] = pltpu.async_copy(rb, acc.at[didx.at[uu]], sem_s[cur], add=True)
    sc[SUPC - 4].wait()
    sc[SUPC - 3].wait()
    sc[SUPC - 2].wait()
    sc[SUPC - 1].wait()

  def zero_ebuf(i, _):
    for j in range(DH // L):
      ebuf[i, pl.ds(j * L, L)] = zero
    return 0
  lax.fori_loop(0, BLK, zero_ebuf, 0)
  for b in range(NBLK):
    base = rbase + b * BLK
    pltpu.sync_copy(ebuf, acc.at[pl.ds(base, BLK)])
    pltpu.sync_copy(ebuf, xacc_hbm.at[c, pl.ds(base, BLK)])
    pltpu.sync_copy(x2.at[c, pl.ds(base, BLK)], ebuf2)
    pltpu.sync_copy(ebuf2, h_sp.at[pl.ds(base, BLK)])
  plsc.subcore_barrier()

  def hop(_, carry):
    fire_idx(0, 0)

    def pair(v, _):
      fire_idx(2 * v + 1, 1)
      process(0)
      fire_idx(2 * v + 2, 0)
      process(1)
      return 0
    lax.fori_loop(0, NSUP // 2 - 1, pair, 0)
    fire_idx(NSUP - 1, 1)
    process(0)
    process(1)
    plsc.subcore_barrier()

    for b in range(NBLK):
      base = rbase + b * BLK
      pltpu.sync_copy(acc.at[pl.ds(base, BLK)], ebuf)
      pltpu.sync_copy(xacc_hbm.at[c, pl.ds(base, BLK)], ebuf2)

      def scale(i, _):
        for j in range(DH // L):
          sl = pl.ds(j * L, L)
          v = ebuf[i, sl] * (1.0 - ALPHA)
          ebuf[i, sl] = v
          ebuf2[i, sl] = ebuf2[i, sl] + v
        return 0
      lax.fori_loop(0, BLK, scale, 0)
      pltpu.sync_copy(ebuf, h_sp.at[pl.ds(base, BLK)])
      pltpu.sync_copy(ebuf2, xacc_hbm.at[c, pl.ds(base, BLK)])

      def rezero(i, _):
        for j in range(DH // L):
          ebuf[i, pl.ds(j * L, L)] = zero
        return 0
      lax.fori_loop(0, BLK, rezero, 0)
      pltpu.sync_copy(ebuf, acc.at[pl.ds(base, BLK)])
    plsc.subcore_barrier()
    return carry
  lax.fori_loop(0, K, hop, 0)

  for b in range(NBLK):
    base = rbase + b * BLK
    pltpu.sync_copy(x2.at[c, pl.ds(base, BLK)], ebuf)
    pltpu.sync_copy(xacc_hbm.at[c, pl.ds(base, BLK)], ebuf2)

    def fin(i, _):
      for j in range(DH // L):
        sl = pl.ds(j * L, L)
        ebuf[i, sl] = (ebuf2[i, sl] * (1.0 / K)
                       + ebuf[i, sl] * (1.0 / K + ALPHA))
      return 0
    lax.fori_loop(0, BLK, fin, 0)
    pltpu.sync_copy(ebuf, out2.at[c, pl.ds(base, BLK)])


@jax.jit
def _run(x2, src_e, dst_e, w_e):
  mesh = plsc.VectorSubcoreMesh(core_axis_name="c", subcore_axis_name="s")
  f = pl.kernel(
      _body,
      out_type=(
          jax.ShapeDtypeStruct((NC, NP, DH), jnp.float32),
          jax.ShapeDtypeStruct((NC, NP, DH), jnp.float32),
      ),
      mesh=mesh,
      compiler_params=pltpu.CompilerParams(use_tc_tiling_on_sc=False),
      scratch_types=[
          pltpu.VMEM_SHARED((NP, DH), jnp.float32),
          pltpu.VMEM_SHARED((NP, DH), jnp.float32),
          pltpu.VMEM((SUPC, CH), jnp.int32),
          pltpu.VMEM((SUPC, CH), jnp.int32),
          pltpu.VMEM((SUPC, CH), jnp.int32),
          pltpu.VMEM((SUPC, CH), jnp.int32),
          pltpu.VMEM((SUPC, CH), jnp.float32),
          pltpu.VMEM((SUPC, CH), jnp.float32),
          pltpu.VMEM((CH, DH), jnp.float32),
          pltpu.VMEM((CH, DH), jnp.float32),
          pltpu.VMEM((CH, DH), jnp.float32),
          pltpu.VMEM((CH, DH), jnp.float32),
          pltpu.VMEM((BLK, DH), jnp.float32),
          pltpu.VMEM((BLK, DH), jnp.float32),
          pltpu.SemaphoreType.DMA,
          pltpu.SemaphoreType.DMA,
          pltpu.SemaphoreType.DMA,
          pltpu.SemaphoreType.DMA,
          pltpu.SemaphoreType.DMA,
          pltpu.SemaphoreType.DMA,
          pltpu.SemaphoreType.DMA,
          pltpu.SemaphoreType.DMA,
          pltpu.SemaphoreType.DMA,
          pltpu.SemaphoreType.DMA,
      ],
  )
  out2, _ = f(x2, src_e, dst_e, w_e)
  return out2


def kernel(x, edge_index, edge_weight):
  x = x.astype(jnp.float32)
  src = edge_index[0].astype(jnp.int32)
  dst = edge_index[1].astype(jnp.int32)
  w = edge_weight.astype(jnp.float32)
  pad = PE - E
  src_e = jnp.concatenate([src, jnp.zeros((pad,), jnp.int32)]).reshape(
      NS, NSUP, SUPC, CH)
  dst_e = jnp.concatenate([dst, jnp.zeros((pad,), jnp.int32)]).reshape(
      NS, NSUP, SUPC, CH)
  w_e = jnp.concatenate([w, jnp.zeros((pad,), jnp.float32)]).reshape(
      NS, NSUP, SUPC, CH)
  x2 = x.reshape(N, NC, DH).transpose(1, 0, 2)
  x2 = jnp.concatenate(
      [x2, jnp.zeros((NC, NP - N, DH), jnp.float32)], axis=1)
  out2 = _run(x2, src_e, dst_e, w_e)
  return out2[:, :N].transpose(1, 0, 2).reshape(N, D)
